# baseline (speedup 1.0000x reference)
import functools

import jax
import jax.numpy as jnp
from jax.experimental import pallas as pl

BLOCK_M = 8192


def _body(x_ref, w_ref, b_ref, o_ref):
    t = jax.lax.dot_general(
        w_ref[...], x_ref[...],
        dimension_numbers=(((0,), (1,)), ((), ())),
        preferred_element_type=jnp.float32,
    )
    o_ref[...] = t + b_ref[...]


@functools.partial(jax.jit, static_argnames=())
def kernel(features, indices, W, b):
    del indices
    m, c_in = features.shape
    c_out = W.shape[1]
    block_m = min(BLOCK_M, m)
    grid = (pl.cdiv(m, block_m),)
    out_t = pl.pallas_call(
        _body,
        grid=grid,
        in_specs=[
            pl.BlockSpec((block_m, c_in), lambda i: (i, 0)),
            pl.BlockSpec((c_in, c_out), lambda i: (0, 0)),
            pl.BlockSpec((c_out, 1), lambda i: (0, 0)),
        ],
        out_specs=pl.BlockSpec((c_out, block_m), lambda i: (0, i)),
        out_shape=jax.ShapeDtypeStruct((c_out, m), jnp.float32),
    )(features, W, b.reshape(c_out, 1))
    return out_t.T


# speedup vs baseline: 1.1196x; 1.1196x over previous
import functools

import jax
import jax.numpy as jnp
from jax.experimental import pallas as pl

BLOCK_M = 16384


def _body(x_ref, w_ref, b_ref, o_ref):
    t = jax.lax.dot_general(
        w_ref[...], x_ref[...],
        dimension_numbers=(((0,), (1,)), ((), ())),
        preferred_element_type=jnp.float32,
    )
    o_ref[...] = t + b_ref[...]


@functools.partial(jax.jit, static_argnames=())
def kernel(features, indices, W, b):
    del indices
    m, c_in = features.shape
    c_out = W.shape[1]
    block_m = min(BLOCK_M, m)
    grid = (pl.cdiv(m, block_m),)
    out_t = pl.pallas_call(
        _body,
        grid=grid,
        in_specs=[
            pl.BlockSpec((block_m, c_in), lambda i: (i, 0)),
            pl.BlockSpec((c_in, c_out), lambda i: (0, 0)),
            pl.BlockSpec((c_out, 1), lambda i: (0, 0)),
        ],
        out_specs=pl.BlockSpec((c_out, block_m), lambda i: (0, i)),
        out_shape=jax.ShapeDtypeStruct((c_out, m), jnp.float32),
    )(features, W, b.reshape(c_out, 1))
    return out_t.T
